# baseline (speedup 1.0000x reference)
import functools

import jax
import jax.numpy as jnp
from jax import lax
from jax.experimental import pallas as pl
from jax.experimental.pallas import tpu as pltpu
from jax.experimental.pallas import tpu_sc as plsc

EMBED_DIM = 32
OUT_DIM = 32
PACK = 128 // EMBED_DIM

_info = plsc.get_sparse_core_info()
_NC = _info.num_cores
_NS = _info.num_subcores
_NW = _NC * _NS

_CH = 104
_NBUF = 8


@functools.partial(jax.jit, static_argnums=(2, 3))
def _sc_gather(idx3, table, B, NNZ):
    R = B * NNZ
    rows_per_w = R // _NW
    n_chunks = rows_per_w // _CH
    n_groups = n_chunks // _NBUF
    elems_per_ch = _CH // NNZ
    elems_per_w = rows_per_w // NNZ
    mesh = plsc.VectorSubcoreMesh(core_axis_name="c", subcore_axis_name="s")

    @functools.partial(
        pl.kernel,
        out_type=jax.ShapeDtypeStruct((B, NNZ, EMBED_DIM), jnp.float32),
        mesh=mesh,
        scratch_types=[
            pltpu.VMEM((n_chunks, _CH), jnp.int32),
        ] + [pltpu.VMEM((_CH, EMBED_DIM), jnp.float32) for _ in range(_NBUF)]
          + [pltpu.SemaphoreType.DMA, pltpu.SemaphoreType.DMA],
        compiler_params=pltpu.CompilerParams(use_tc_tiling_on_sc=False),
    )
    def gather_kernel(idx_hbm, table_hbm, out_hbm, idx_v, *rest):
        bufs = rest[:_NBUF]
        gsem, wsem = rest[_NBUF], rest[_NBUF + 1]
        wid = lax.axis_index("s") * _NC + lax.axis_index("c")
        ebase = wid * elems_per_w
        pltpu.sync_copy(idx_hbm.at[wid], idx_v)

        def group(g, _):
            j0 = g * _NBUF
            gathers = []
            for t in range(_NBUF):
                gathers.append(pltpu.async_copy(
                    table_hbm.at[idx_v.at[j0 + t]], bufs[t], gsem))
            writes = []
            for t in range(_NBUF):
                gathers[t].wait()
                e0 = ebase + (j0 + t) * elems_per_ch
                for q in range(elems_per_ch):
                    writes.append(pltpu.async_copy(
                        bufs[t].at[pl.ds(q * NNZ, NNZ)],
                        out_hbm.at[e0 + q], wsem))
            for t in range(_NBUF):
                for q in range(elems_per_ch):
                    writes[t * elems_per_ch + q].wait()
            return 0

        lax.fori_loop(0, n_groups, group, 0)

    return gather_kernel(idx3, table)


def _table_body(e_ref, w4k_ref, b4_ref, t_ref):
    k = pl.program_id(1)
    acc = jnp.dot(e_ref[...], w4k_ref[...], preferred_element_type=jnp.float32)

    @pl.when(k == 0)
    def _():
        t_ref[...] = acc

    @pl.when(k > 0)
    def _():
        t_ref[...] += acc

    @pl.when(k == PACK - 1)
    def _():
        t_ref[...] = jnp.maximum(t_ref[...] + b4_ref[...], 0.0)


@functools.partial(jax.jit, static_argnums=(3,))
def _tc_transform_table(emb, W4, b4, V):
    BLK = 2000
    V4 = V // PACK
    nq = V4 // BLK
    return pl.pallas_call(
        _table_body,
        grid=(nq, PACK),
        in_specs=[
            pl.BlockSpec((BLK, EMBED_DIM), lambda i, k: (k * nq + i, 0)),
            pl.BlockSpec((EMBED_DIM, PACK * OUT_DIM), lambda i, k: (k, 0)),
            pl.BlockSpec((1, PACK * OUT_DIM), lambda i, k: (0, 0)),
        ],
        out_specs=pl.BlockSpec((BLK, PACK * OUT_DIM), lambda i, k: (i, 0)),
        out_shape=jax.ShapeDtypeStruct((V4, PACK * OUT_DIM), jnp.float32),
    )(emb, W4, b4)


def kernel(x, embedding, W, b):
    B, NNZ = x.shape
    V, D = embedding.shape
    R = B * NNZ
    rows_per_w = R // _NW
    n_chunks = rows_per_w // _CH
    V4 = V // PACK

    W4 = jnp.kron(jnp.eye(PACK, dtype=W.dtype), W)
    b4 = jnp.tile(b, PACK).reshape(1, PACK * OUT_DIM)
    t128 = _tc_transform_table(embedding, W4, b4, V)
    table = t128.reshape(V, D)

    xi = x.astype(jnp.int32)
    perm = (xi % V4) * PACK + xi // V4
    idx3 = perm.reshape(_NW, n_chunks, _CH)
    return _sc_gather(idx3, table, B, NNZ)


# speedup vs baseline: 1.3671x; 1.3671x over previous
import functools

import jax
import jax.numpy as jnp
from jax import lax
from jax.experimental import pallas as pl
from jax.experimental.pallas import tpu as pltpu
from jax.experimental.pallas import tpu_sc as plsc

EMBED_DIM = 32
OUT_DIM = 32
PACK = 128 // EMBED_DIM

_info = plsc.get_sparse_core_info()
_NC = _info.num_cores
_NS = _info.num_subcores
_NW = _NC * _NS

_CH = 104
_NBUF = 8


@functools.partial(jax.jit, static_argnums=(2, 3))
def _sc_gather(idx3, table, B, NNZ):
    R = B * NNZ
    rows_per_w = R // _NW
    n_chunks = rows_per_w // _CH
    n_groups = n_chunks // _NBUF
    elems_per_ch = _CH // NNZ
    elems_per_w = rows_per_w // NNZ
    mesh = plsc.VectorSubcoreMesh(core_axis_name="c", subcore_axis_name="s")

    @functools.partial(
        pl.kernel,
        out_type=jax.ShapeDtypeStruct((B, NNZ, EMBED_DIM), jnp.float32),
        mesh=mesh,
        scratch_types=[
            pltpu.VMEM((n_chunks, _CH), jnp.int32),
        ] + [pltpu.VMEM((_CH, EMBED_DIM), jnp.float32) for _ in range(_NBUF)]
          + [pltpu.SemaphoreType.DMA, pltpu.SemaphoreType.DMA],
        compiler_params=pltpu.CompilerParams(use_tc_tiling_on_sc=False),
    )
    def gather_kernel(idx_hbm, table_hbm, out_hbm, idx_v, *rest):
        bufs = rest[:_NBUF]
        gsem, wsem = rest[_NBUF], rest[_NBUF + 1]
        wid = lax.axis_index("s") * _NC + lax.axis_index("c")
        ebase = wid * elems_per_w
        pltpu.sync_copy(idx_hbm.at[wid], idx_v)

        def group(g, _):
            j0 = g * _NBUF
            gathers = []
            for t in range(_NBUF):
                gathers.append(pltpu.async_copy(
                    table_hbm.at[idx_v.at[j0 + t]], bufs[t], gsem))
            writes = []
            for t in range(_NBUF):
                gathers[t].wait()
                e0 = ebase + (j0 + t) * elems_per_ch
                for q in range(elems_per_ch):
                    writes.append(pltpu.async_copy(
                        bufs[t].at[pl.ds(q * NNZ, NNZ)],
                        out_hbm.at[e0 + q], wsem))
            for t in range(_NBUF):
                for q in range(elems_per_ch):
                    writes[t * elems_per_ch + q].wait()
            return 0

        lax.fori_loop(0, n_groups, group, 0)

    return gather_kernel(idx3, table)


def _table_body(e0_ref, e1_ref, e2_ref, e3_ref, w4_ref, b4_ref, t_ref):
    x4 = jnp.concatenate(
        [e0_ref[...], e1_ref[...], e2_ref[...], e3_ref[...]], axis=-1)
    y = jnp.dot(x4, w4_ref[...], preferred_element_type=jnp.float32)
    t_ref[...] = jnp.maximum(y + b4_ref[...], 0.0)


@functools.partial(jax.jit, static_argnums=(3,))
def _tc_transform_table(emb, W4, b4, V):
    BLK = 5000
    V4 = V // PACK
    nq = V4 // BLK
    quarter_spec = [
        pl.BlockSpec((BLK, EMBED_DIM), (lambda i, k=k: (k * nq + i, 0)))
        for k in range(PACK)
    ]
    return pl.pallas_call(
        _table_body,
        grid=(nq,),
        in_specs=quarter_spec + [
            pl.BlockSpec((PACK * EMBED_DIM, PACK * OUT_DIM), lambda i: (0, 0)),
            pl.BlockSpec((1, PACK * OUT_DIM), lambda i: (0, 0)),
        ],
        out_specs=pl.BlockSpec((BLK, PACK * OUT_DIM), lambda i: (i, 0)),
        out_shape=jax.ShapeDtypeStruct((V4, PACK * OUT_DIM), jnp.float32),
        compiler_params=pltpu.CompilerParams(
            allow_input_fusion=[True, True, True, True, False, False]),
    )(emb, emb, emb, emb, W4, b4)


def kernel(x, embedding, W, b):
    B, NNZ = x.shape
    V, D = embedding.shape
    R = B * NNZ
    rows_per_w = R // _NW
    n_chunks = rows_per_w // _CH
    V4 = V // PACK

    W4 = jnp.kron(jnp.eye(PACK, dtype=W.dtype), W)
    b4 = jnp.tile(b, PACK).reshape(1, PACK * OUT_DIM)
    t128 = _tc_transform_table(embedding, W4, b4, V)
    table = t128.reshape(V, D)

    xi = x.astype(jnp.int32)
    perm = (xi % V4) * PACK + xi // V4
    idx3 = perm.reshape(_NW, n_chunks, _CH)
    return _sc_gather(idx3, table, B, NNZ)
